# baseline (speedup 1.0000x reference)
import functools

import jax
import jax.numpy as jnp
from jax import lax
from jax.experimental import pallas as pl
from jax.experimental.pallas import tpu as pltpu
from jax.experimental.pallas import tpu_sc as plsc

N = 10000
E = 320000
D = 128
HD = D // 2
S = 8

NC = 2
NS = 16
EPT = E // NS
C = 128
NCHUNK = 160
EPTP = NCHUNK * C
NP = 10240
RPT = NP // NS

CW = 8
_B = 128
_NB = RPT // _B

_SQRT_HALF = 0.7071067811865476


NBUF = 5


def _sc_agg_body(with_cnt, x2_hbm, src3_hbm, dst3_hbm, zeros_bd, zeros_n16,
                 ones_c16, acc_hbm, cnt_hbm, src_v, dst_v, rows_v, ones_v,
                 buf_v, cbuf_v, acc_sh, cnt_sh, gsem, ssem, csem, sem):
    _NOUTER = NCHUNK // NBUF
    cid = lax.axis_index("c")
    sid = lax.axis_index("s")

    idx_a = pltpu.async_copy(src3_hbm.at[cid, sid], src_v, sem)
    idx_b = pltpu.async_copy(dst3_hbm.at[sid], dst_v, sem)

    pltpu.sync_copy(zeros_bd, buf_v)
    for j in range(_NB):
        pltpu.sync_copy(buf_v, acc_sh.at[pl.ds(sid * RPT + j * _B, _B)])
    if with_cnt:
        pltpu.sync_copy(zeros_n16, cbuf_v)
        pltpu.sync_copy(cbuf_v, cnt_sh.at[pl.ds(sid * RPT, RPT)])
        pltpu.sync_copy(ones_c16, ones_v)
    idx_a.wait()
    idx_b.wait()

    for b in range(NBUF):
        pltpu.async_copy(x2_hbm.at[src_v.at[b]], rows_v.at[b], gsem.at[b])
    plsc.subcore_barrier()

    def outer(i, carry):
        for b in range(NBUF):
            c = i * NBUF + b
            pltpu.make_async_copy(x2_hbm.at[src_v.at[b]], rows_v.at[b],
                                  gsem.at[b]).wait()
            if with_cnt:
                @pl.when(cid == (0 if b < 3 else 1))
                def _():
                    @pl.when(i > 0)
                    def _():
                        pltpu.make_async_copy(ones_v, cnt_sh.at[dst_v.at[c]],
                                              csem.at[b]).wait()
                    pltpu.async_copy(ones_v, cnt_sh.at[dst_v.at[c]],
                                     csem.at[b], add=True)
            pltpu.async_copy(rows_v.at[b], acc_sh.at[dst_v.at[c]],
                             ssem.at[b], add=True)
            @pl.when(i < _NOUTER - 1)
            def _():
                pltpu.make_async_copy(rows_v.at[b], acc_sh.at[dst_v.at[c]],
                                      ssem.at[b]).wait()
                pltpu.async_copy(x2_hbm.at[src_v.at[c + NBUF]], rows_v.at[b],
                                 gsem.at[b])
        return carry

    lax.fori_loop(0, _NOUTER, outer, 0)
    for b in range(NBUF):
        c = (_NOUTER - 1) * NBUF + b
        pltpu.make_async_copy(rows_v.at[b], acc_sh.at[dst_v.at[c]],
                              ssem.at[b]).wait()
        if with_cnt:
            @pl.when(cid == (0 if b < 3 else 1))
            def _():
                pltpu.make_async_copy(ones_v, cnt_sh.at[dst_v.at[c]],
                                      csem.at[b]).wait()
    plsc.subcore_barrier()

    base = cid * NP + sid * RPT
    for j in range(_NB):
        pltpu.sync_copy(acc_sh.at[pl.ds(sid * RPT + j * _B, _B)], buf_v)
        pltpu.sync_copy(buf_v, acc_hbm.at[pl.ds(base + j * _B, _B)])
    if with_cnt:
        pltpu.sync_copy(cnt_sh.at[pl.ds(sid * RPT, RPT)], cbuf_v)
        pltpu.sync_copy(cbuf_v, cnt_hbm.at[pl.ds(base, RPT)])


def _make_sc_agg(with_cnt):
    out_type = [jax.ShapeDtypeStruct((NC * NP, HD), jnp.bfloat16)]
    if with_cnt:
        out_type.append(jax.ShapeDtypeStruct((NC * NP, CW), jnp.float32))
    mesh = plsc.VectorSubcoreMesh(core_axis_name="c", subcore_axis_name="s",
                                  num_cores=NC, num_subcores=NS)
    if with_cnt:
        body = functools.partial(_sc_agg_body, True)
    else:
        def body(x2_hbm, src3_hbm, dst3_hbm, zeros_bd, zeros_n16, ones_c16,
                 acc_hbm, src_v, dst_v, rows_v, ones_v, buf_v, cbuf_v,
                 acc_sh, cnt_sh, gsem, ssem, csem, sem):
            _sc_agg_body(False, x2_hbm, src3_hbm, dst3_hbm, zeros_bd,
                         zeros_n16, ones_c16, acc_hbm, None, src_v, dst_v,
                         rows_v, ones_v, buf_v, cbuf_v, acc_sh, cnt_sh,
                         gsem, ssem, csem, sem)
    return pl.kernel(
        body,
        out_type=tuple(out_type) if with_cnt else out_type[0],
        mesh=mesh,
        scratch_types=[
            pltpu.VMEM((NCHUNK, C), jnp.int32),
            pltpu.VMEM((NCHUNK, C), jnp.int32),
            pltpu.VMEM((NBUF, C, HD), jnp.bfloat16),
            pltpu.VMEM((C, CW), jnp.float32),
            pltpu.VMEM((_B, HD), jnp.bfloat16),
            pltpu.VMEM((RPT, CW), jnp.float32),
            pltpu.VMEM_SHARED((NP, HD), jnp.bfloat16),
            pltpu.VMEM_SHARED((NP, CW), jnp.float32),
            pltpu.SemaphoreType.DMA((NBUF,)),
            pltpu.SemaphoreType.DMA((NBUF,)),
            pltpu.SemaphoreType.DMA((NBUF,)),
            pltpu.SemaphoreType.DMA,
        ],
        compiler_params=pltpu.CompilerParams(use_tc_tiling_on_sc=False),
        name="sc_segment_mean" + ("_cnt" if with_cnt else ""),
    )


def _gelu(h):
    return 0.5 * h * (1.0 + lax.erf(h * _SQRT_HALF))


def _dot_t(a, w):
    return lax.dot_general(a, w, (((1,), (1,)), ((), ())),
                           preferred_element_type=jnp.float32)


def _mean_halves(acc_ref, cnt_ref):
    cnt = jnp.maximum(cnt_ref[0][:, :1] + cnt_ref[1][:, :1], 1.0)
    return (acc_ref[0].astype(jnp.float32) / cnt,
            acc_ref[1].astype(jnp.float32) / cnt)


def _combine_body(acc_ref, cnt_ref, x_ref, wll_ref, wlr_ref, bl_ref, wr_ref,
                  out_ref, outb_ref):
    meanl, meanr = _mean_halves(acc_ref, cnt_ref)
    h = (_dot_t(meanl, wll_ref[...]) + _dot_t(meanr, wlr_ref[...])
         + bl_ref[...] + _dot_t(x_ref[...], wr_ref[...]))
    h = _gelu(h)
    out_ref[...] = h
    outb_ref[...] = h.astype(jnp.bfloat16)


def _tail_body(acc_ref, cnt_ref, h_ref, wll_ref, wlr_ref, bl_ref, wr_ref,
               wc1_ref, bc1_ref, wc2_ref, bc2_ref, out_ref):
    meanl, meanr = _mean_halves(acc_ref, cnt_ref)
    h = (_dot_t(meanl, wll_ref[...]) + _dot_t(meanr, wlr_ref[...])
         + bl_ref[...] + _dot_t(h_ref[...], wr_ref[...]))
    h = _gelu(h)
    h = _gelu(_dot_t(h, wc1_ref[...]) + bc1_ref[...])
    logits = _dot_t(h, wc2_ref[...]) + bc2_ref[...]
    m = jnp.max(logits, axis=1, keepdims=True)
    e = jnp.exp(logits - m)
    out_ref[...] = e / jnp.sum(e, axis=1, keepdims=True)


_R = 2000
_GRID = N // _R

_full2 = lambda shape: pl.BlockSpec(shape, lambda i: (0, 0))
_acc_spec = pl.BlockSpec((NC, _R, HD), lambda i: (0, i, 0))
_cnt_spec = pl.BlockSpec((NC, _R, CW), lambda i: (0, i, 0))
_row_spec = pl.BlockSpec((_R, D), lambda i: (i, 0))


def _tc_combine(acc, cnt, x, wll, wlr, bl, wr):
    return pl.pallas_call(
        _combine_body,
        grid=(_GRID,),
        in_specs=[_acc_spec, _cnt_spec, _row_spec,
                  _full2((D, HD)), _full2((D, HD)), _full2((1, D)),
                  _full2((D, D))],
        out_specs=[_row_spec, _row_spec],
        out_shape=[jax.ShapeDtypeStruct((N, D), jnp.float32),
                   jax.ShapeDtypeStruct((N, D), jnp.bfloat16)],
    )(acc, cnt, x, wll, wlr, bl, wr)


def _tc_tail(acc, cnt, h, wll, wlr, bl, wr, wc1, bc1, wc2, bc2):
    return pl.pallas_call(
        _tail_body,
        grid=(_GRID,),
        in_specs=[_acc_spec, _cnt_spec, _row_spec,
                  _full2((D, HD)), _full2((D, HD)), _full2((1, D)),
                  _full2((D, D)), _full2((D, D)), _full2((1, D)),
                  _full2((S, D)), _full2((1, S))],
        out_specs=pl.BlockSpec((_R, S), lambda i: (i, 0)),
        out_shape=jax.ShapeDtypeStruct((N, S), jnp.float32),
    )(acc, cnt, h, wll, wlr, bl, wr, wc1, bc1, wc2, bc2)


_sc_agg_cnt = _make_sc_agg(True)
_sc_agg = _make_sc_agg(False)


def kernel(x, edge_index, Wl1, bl1, Wr1, Wl2, bl2, Wr2, Wc1, bc1, Wc2, bc2):
    src = edge_index[0].astype(jnp.int32)
    dst = edge_index[1].astype(jnp.int32)
    cvec = jnp.arange(NC, dtype=jnp.int32).reshape(NC, 1, 1)
    src3 = jnp.concatenate(
        [2 * src.reshape(NS, EPT)[None] + cvec,
         jnp.zeros((NC, NS, EPTP - EPT), jnp.int32)], axis=2)
    src3 = src3.reshape(NC, NS, NCHUNK, C)
    trash = (NP - NS + jnp.arange(NS, dtype=jnp.int32))[:, None]
    dst3 = jnp.concatenate(
        [dst.reshape(NS, EPT),
         jnp.broadcast_to(trash, (NS, EPTP - EPT))], axis=1)
    dst3 = dst3.reshape(NS, NCHUNK, C)
    zeros_bd = jnp.zeros((_B, HD), jnp.bfloat16)
    zeros_n16 = jnp.zeros((RPT, CW), jnp.float32)
    ones_c16 = jnp.ones((C, CW), jnp.float32)

    acc1, cnt = _sc_agg_cnt(x.astype(jnp.bfloat16).reshape(2 * N, HD),
                            src3, dst3, zeros_bd, zeros_n16, ones_c16)
    acc1 = acc1.reshape(NC, NP, HD)
    cnt = cnt.reshape(NC, NP, CW)
    h1, h1b = _tc_combine(acc1, cnt, x, Wl1[:, :HD], Wl1[:, HD:],
                          bl1.reshape(1, D), Wr1)

    acc2 = _sc_agg(h1b.reshape(2 * N, HD), src3, dst3,
                   zeros_bd, zeros_n16, ones_c16)
    acc2 = acc2.reshape(NC, NP, HD)
    return _tc_tail(acc2, cnt, h1, Wl2[:, :HD], Wl2[:, HD:],
                    bl2.reshape(1, D), Wr2,
                    Wc1, bc1.reshape(1, D), Wc2, bc2.reshape(1, S))


# speedup vs baseline: 2.3104x; 2.3104x over previous
import functools

import jax
import jax.numpy as jnp
from jax import lax
from jax.experimental import pallas as pl
from jax.experimental.pallas import tpu as pltpu
from jax.experimental.pallas import tpu_sc as plsc

N = 10000
E = 320000
D = 128
HD = D // 2
S = 8

NC = 2
NS = 16
EPT = E // NS
NP = 10240
RPT = NP // NS

CW = 8
_B = 128
_NB = RPT // _B

_SQRT_HALF = 0.7071067811865476


NBUF = 5


def _sc_agg_body(with_cnt, C, x2_hbm, src3_hbm, dst3_hbm, zeros_bd, zeros_n16,
                 ones_c16, acc_hbm, cnt_hbm, src_v, dst_v, rows_v, ones_v,
                 buf_v, cbuf_v, acc_sh, cnt_sh, gsem, ssem, csem, sem):
    NCHUNK = EPT // C
    _NOUTER = NCHUNK // NBUF
    cid = lax.axis_index("c")
    sid = lax.axis_index("s")

    idx_a = pltpu.async_copy(src3_hbm.at[cid, sid], src_v, sem)
    idx_b = pltpu.async_copy(dst3_hbm.at[sid], dst_v, sem)

    pltpu.sync_copy(zeros_bd, buf_v)
    for j in range(_NB):
        pltpu.sync_copy(buf_v, acc_sh.at[pl.ds(sid * RPT + j * _B, _B)])
    if with_cnt:
        pltpu.sync_copy(zeros_n16, cbuf_v)
        pltpu.sync_copy(cbuf_v, cnt_sh.at[pl.ds(sid * RPT, RPT)])
        pltpu.sync_copy(ones_c16, ones_v)
    idx_a.wait()
    idx_b.wait()

    for b in range(NBUF):
        pltpu.async_copy(x2_hbm.at[src_v.at[b]], rows_v.at[b], gsem.at[b])
    plsc.subcore_barrier()

    def outer(i, carry):
        for b in range(NBUF):
            c = i * NBUF + b
            pltpu.make_async_copy(x2_hbm.at[src_v.at[b]], rows_v.at[b],
                                  gsem.at[b]).wait()
            if with_cnt:
                @pl.when(cid == (0 if b < 3 else 1))
                def _():
                    @pl.when(i > 0)
                    def _():
                        pltpu.make_async_copy(ones_v, cnt_sh.at[dst_v.at[c]],
                                              csem.at[b]).wait()
                    pltpu.async_copy(ones_v, cnt_sh.at[dst_v.at[c]],
                                     csem.at[b], add=True)
            pltpu.async_copy(rows_v.at[b], acc_sh.at[dst_v.at[c]],
                             ssem.at[b], add=True)
            @pl.when(i < _NOUTER - 1)
            def _():
                pltpu.make_async_copy(rows_v.at[b], acc_sh.at[dst_v.at[c]],
                                      ssem.at[b]).wait()
                pltpu.async_copy(x2_hbm.at[src_v.at[c + NBUF]], rows_v.at[b],
                                 gsem.at[b])
        return carry

    lax.fori_loop(0, _NOUTER, outer, 0)
    for b in range(NBUF):
        c = (_NOUTER - 1) * NBUF + b
        pltpu.make_async_copy(rows_v.at[b], acc_sh.at[dst_v.at[c]],
                              ssem.at[b]).wait()
        if with_cnt:
            @pl.when(cid == (0 if b < 3 else 1))
            def _():
                pltpu.make_async_copy(ones_v, cnt_sh.at[dst_v.at[c]],
                                      csem.at[b]).wait()
    plsc.subcore_barrier()

    base = cid * NP + sid * RPT
    for j in range(_NB):
        pltpu.sync_copy(acc_sh.at[pl.ds(sid * RPT + j * _B, _B)], buf_v)
        pltpu.sync_copy(buf_v, acc_hbm.at[pl.ds(base + j * _B, _B)])
    if with_cnt:
        pltpu.sync_copy(cnt_sh.at[pl.ds(sid * RPT, RPT)], cbuf_v)
        pltpu.sync_copy(cbuf_v, cnt_hbm.at[pl.ds(base, RPT)])


def _make_sc_agg(with_cnt, C):
    NCHUNK = EPT // C
    out_type = [jax.ShapeDtypeStruct((NC * NP, HD), jnp.bfloat16)]
    if with_cnt:
        out_type.append(jax.ShapeDtypeStruct((NC * NP, CW), jnp.float32))
    mesh = plsc.VectorSubcoreMesh(core_axis_name="c", subcore_axis_name="s",
                                  num_cores=NC, num_subcores=NS)
    if with_cnt:
        body = functools.partial(_sc_agg_body, True, C)
    else:
        def body(x2_hbm, src3_hbm, dst3_hbm, zeros_bd, zeros_n16, ones_c16,
                 acc_hbm, src_v, dst_v, rows_v, ones_v, buf_v, cbuf_v,
                 acc_sh, cnt_sh, gsem, ssem, csem, sem):
            _sc_agg_body(False, C, x2_hbm, src3_hbm, dst3_hbm, zeros_bd,
                         zeros_n16, ones_c16, acc_hbm, None, src_v, dst_v,
                         rows_v, ones_v, buf_v, cbuf_v, acc_sh, cnt_sh,
                         gsem, ssem, csem, sem)
    return pl.kernel(
        body,
        out_type=tuple(out_type) if with_cnt else out_type[0],
        mesh=mesh,
        scratch_types=[
            pltpu.VMEM((NCHUNK, C), jnp.int32),
            pltpu.VMEM((NCHUNK, C), jnp.int32),
            pltpu.VMEM((NBUF, C, HD), jnp.bfloat16),
            pltpu.VMEM((C, CW), jnp.float32),
            pltpu.VMEM((_B, HD), jnp.bfloat16),
            pltpu.VMEM((RPT, CW), jnp.float32),
            pltpu.VMEM_SHARED((NP, HD), jnp.bfloat16),
            pltpu.VMEM_SHARED((NP, CW), jnp.float32),
            pltpu.SemaphoreType.DMA((NBUF,)),
            pltpu.SemaphoreType.DMA((NBUF,)),
            pltpu.SemaphoreType.DMA((NBUF,)),
            pltpu.SemaphoreType.DMA,
        ],
        compiler_params=pltpu.CompilerParams(use_tc_tiling_on_sc=False),
        name="sc_segment_mean" + ("_cnt" if with_cnt else ""),
    )


def _gelu(h):
    return 0.5 * h * (1.0 + lax.erf(h * _SQRT_HALF))


def _dot_t(a, w):
    return lax.dot_general(a, w, (((1,), (1,)), ((), ())),
                           preferred_element_type=jnp.float32)


def _mean_halves(acc_ref, cnt_ref):
    cnt = jnp.maximum(cnt_ref[0][:, :1] + cnt_ref[1][:, :1], 1.0)
    return (acc_ref[0].astype(jnp.float32) / cnt,
            acc_ref[1].astype(jnp.float32) / cnt)


def _combine_body(acc_ref, cnt_ref, x_ref, wll_ref, wlr_ref, bl_ref, wr_ref,
                  out_ref):
    meanl, meanr = _mean_halves(acc_ref, cnt_ref)
    h = (_dot_t(meanl, wll_ref[...]) + _dot_t(meanr, wlr_ref[...])
         + bl_ref[...] + _dot_t(x_ref[...], wr_ref[...]))
    out_ref[...] = _gelu(h)


def _tail_body(acc_ref, cnt_ref, h_ref, wll_ref, wlr_ref, bl_ref, wr_ref,
               wc1_ref, bc1_ref, wc2_ref, bc2_ref, out_ref):
    meanl, meanr = _mean_halves(acc_ref, cnt_ref)
    h = (_dot_t(meanl, wll_ref[...]) + _dot_t(meanr, wlr_ref[...])
         + bl_ref[...] + _dot_t(h_ref[...], wr_ref[...]))
    h = _gelu(h)
    h = _gelu(_dot_t(h, wc1_ref[...]) + bc1_ref[...])
    logits = _dot_t(h, wc2_ref[...]) + bc2_ref[...]
    m = jnp.max(logits, axis=1, keepdims=True)
    e = jnp.exp(logits - m)
    out_ref[...] = e / jnp.sum(e, axis=1, keepdims=True)


_R = 2000
_GRID = N // _R

_full2 = lambda shape: pl.BlockSpec(shape, lambda i: (0, 0))
_acc_spec = pl.BlockSpec((NC, _R, HD), lambda i: (0, i, 0))
_cnt_spec = pl.BlockSpec((NC, _R, CW), lambda i: (0, i, 0))
_row_spec = pl.BlockSpec((_R, D), lambda i: (i, 0))


def _tc_combine(acc, cnt, x, wll, wlr, bl, wr):
    return pl.pallas_call(
        _combine_body,
        grid=(_GRID,),
        in_specs=[_acc_spec, _cnt_spec, _row_spec,
                  _full2((D, HD)), _full2((D, HD)), _full2((1, D)),
                  _full2((D, D))],
        out_specs=_row_spec,
        out_shape=jax.ShapeDtypeStruct((N, D), jnp.float32),
    )(acc, cnt, x, wll, wlr, bl, wr)


def _tc_tail(acc, cnt, h, wll, wlr, bl, wr, wc1, bc1, wc2, bc2):
    return pl.pallas_call(
        _tail_body,
        grid=(_GRID,),
        in_specs=[_acc_spec, _cnt_spec, _row_spec,
                  _full2((D, HD)), _full2((D, HD)), _full2((1, D)),
                  _full2((D, D)), _full2((D, D)), _full2((1, D)),
                  _full2((S, D)), _full2((1, S))],
        out_specs=pl.BlockSpec((_R, S), lambda i: (i, 0)),
        out_shape=jax.ShapeDtypeStruct((N, S), jnp.float32),
    )(acc, cnt, h, wll, wlr, bl, wr, wc1, bc1, wc2, bc2)


C1 = 125
C2 = 125
_sc_agg_cnt = _make_sc_agg(True, C1)
_sc_agg = _make_sc_agg(False, C2)


def kernel(x, edge_index, Wl1, bl1, Wr1, Wl2, bl2, Wr2, Wc1, bc1, Wc2, bc2):
    src = edge_index[0].astype(jnp.int32)
    dst = edge_index[1].astype(jnp.int32)
    src2 = jnp.stack([2 * src, 2 * src + 1])
    src3a = src2.reshape(NC, NS, EPT // C1, C1)
    dst3a = dst.reshape(NS, EPT // C1, C1)
    src3b = src2.reshape(NC, NS, EPT // C2, C2)
    dst3b = dst.reshape(NS, EPT // C2, C2)
    zeros_bd = jnp.zeros((_B, HD), jnp.bfloat16)
    zeros_n16 = jnp.zeros((RPT, CW), jnp.float32)
    ones_c16 = jnp.ones((C1, CW), jnp.float32)

    acc1, cnt = _sc_agg_cnt(x.astype(jnp.bfloat16).reshape(2 * N, HD),
                            src3a, dst3a, zeros_bd, zeros_n16, ones_c16)
    acc1 = acc1.reshape(NC, NP, HD)
    cnt = cnt.reshape(NC, NP, CW)
    h1 = _tc_combine(acc1, cnt, x, Wl1[:, :HD], Wl1[:, HD:],
                     bl1.reshape(1, D), Wr1)

    acc2 = _sc_agg(h1.astype(jnp.bfloat16).reshape(2 * N, HD), src3b, dst3b,
                   zeros_bd, zeros_n16, ones_c16)
    acc2 = acc2.reshape(NC, NP, HD)
    return _tc_tail(acc2, cnt, h1, Wl2[:, :HD], Wl2[:, HD:],
                    bl2.reshape(1, D), Wr2,
                    Wc1, bc1.reshape(1, D), Wc2, bc2.reshape(1, S))
